# Initial kernel scaffold; baseline (speedup 1.0000x reference)
import jax
import jax.numpy as jnp
from jax.experimental import pallas as pl


def kernel(edge_attr, W0, W1, W2):
    raise NotImplementedError("write your pallas kernel here")


# speedup vs baseline: 3.1028x; 3.1028x over previous
import functools

import jax
import jax.numpy as jnp
from jax import lax
from jax.experimental import pallas as pl
from jax.experimental.pallas import tpu as pltpu
from jax.experimental.pallas import tpu_sc as plsc

EMB = 16
VOCABS = (5, 6, 2)
NCODES = VOCABS[0] * VOCABS[1] * VOCABS[2]
LANES = 16


@functools.cache
def _build_sc_kernel(E: int):
    NC, NS = 2, 16
    NW = NC * NS
    rows_per_w = E // NW
    CHUNK = 2000
    assert rows_per_w % CHUNK == 0
    nchunks = rows_per_w // CHUNK

    mesh = plsc.VectorSubcoreMesh(core_axis_name="c", subcore_axis_name="s")

    @functools.partial(
        pl.kernel,
        out_type=jax.ShapeDtypeStruct((E * EMB,), jnp.float32),
        mesh=mesh,
        compiler_params=pltpu.CompilerParams(needs_layout_passes=False),
        scratch_types=[
            pltpu.VMEM((VOCABS[0] * EMB,), jnp.float32),
            pltpu.VMEM((VOCABS[1] * EMB,), jnp.float32),
            pltpu.VMEM((VOCABS[2] * EMB,), jnp.float32),
            pltpu.VMEM((NCODES * EMB,), jnp.float32),
            pltpu.VMEM((CHUNK * 3,), jnp.int32),
            pltpu.VMEM((CHUNK * EMB,), jnp.float32),
        ],
    )
    def body(edge_hbm, w0_hbm, w1_hbm, w2_hbm, out_hbm,
             w0_v, w1_v, w2_v, table_v, in_v, out_v):
        wid = lax.axis_index("s") * NC + lax.axis_index("c")
        my_base = wid * rows_per_w

        pltpu.sync_copy(w0_hbm, w0_v)
        pltpu.sync_copy(w1_hbm, w1_v)
        pltpu.sync_copy(w2_hbm, w2_v)
        for a in range(VOCABS[0]):
            for b in range(VOCABS[1]):
                ab = w0_v[pl.ds(a * EMB, EMB)] + w1_v[pl.ds(b * EMB, EMB)]
                for c in range(VOCABS[2]):
                    k = (a * VOCABS[1] + b) * VOCABS[2] + c
                    table_v[pl.ds(k * EMB, EMB)] = ab + w2_v[pl.ds(c * EMB, EMB)]

        iota = lax.iota(jnp.int32, LANES)
        iota3 = iota * 3
        iota16 = iota * EMB

        @pl.loop(0, nchunks)
        def _chunk(g):
            base = my_base + g * CHUNK
            pltpu.sync_copy(edge_hbm.at[pl.ds(base * 3, CHUNK * 3)], in_v)

            @pl.loop(0, CHUNK // LANES)
            def _grp(j):
                idx_a = j * (3 * LANES) + iota3
                a = plsc.load_gather(in_v, [idx_a])
                b = plsc.load_gather(in_v, [idx_a + 1])
                c = plsc.load_gather(in_v, [idx_a + 2])
                code16 = ((a * VOCABS[1] + b) * VOCABS[2] + c) * EMB
                obase = j * (EMB * LANES) + iota16
                for d in range(EMB):
                    v = plsc.load_gather(table_v, [code16 + d])
                    plsc.store_scatter(out_v, [obase + d], v)

            pltpu.sync_copy(out_v, out_hbm.at[pl.ds(base * EMB, CHUNK * EMB)])

    return body


def kernel(edge_attr, W0, W1, W2):
    edge_attr = edge_attr.astype(jnp.int32)
    E = edge_attr.shape[0]
    out_flat = _build_sc_kernel(E)(
        edge_attr.reshape(-1), W0.reshape(-1), W1.reshape(-1), W2.reshape(-1))
    return out_flat.reshape(E, EMB)
